# baseline (speedup 1.0000x reference)
import functools

import jax
import jax.numpy as jnp
from jax import lax
from jax.experimental import pallas as pl
from jax.experimental.pallas import tpu as pltpu
from jax.experimental.pallas import tpu_sc as plsc

_K = 8192
_D = 32
_B = 16384
_TAU = 0.07
_BLOCK_B = 1024
_DP = 128
_L = 16


def _nr_rsqrt(x):
    x = jnp.maximum(x, 1e-35)
    i = plsc.bitcast(x, jnp.int32)
    y = plsc.bitcast(jnp.int32(0x5F3759DF) - (i >> 1), jnp.float32)
    for _ in range(3):
        y = y * (1.5 - 0.5 * x * y * y)
    return y


def _sc_pos_term(ptab_padded, emb_hbm_arg, pos_idx):
    info = plsc.get_sparse_core_info()
    nw = info.num_cores * info.num_subcores
    bpw = _B // nw
    ngroups = bpw // _L
    mesh = plsc.VectorSubcoreMesh(core_axis_name="c", subcore_axis_name="s")

    @functools.partial(
        pl.kernel,
        mesh=mesh,
        out_type=jax.ShapeDtypeStruct((nw, _L), jnp.float32),
        compiler_params=pltpu.CompilerParams(
            needs_layout_passes=False, use_tc_tiling_on_sc=False),
        scratch_types=[
            pltpu.VMEM((bpw,), jnp.int32),
            pltpu.VMEM((bpw, _D), jnp.float32),
            pltpu.VMEM((bpw, _D), jnp.float32),
            pltpu.VMEM((_L,), jnp.float32),
            pltpu.SemaphoreType.DMA,
        ],
    )
    def pos_kernel(table_hbm, emb_hbm, idx_hbm, out_hbm,
                   idx_v, rows_v, e_v, acc_v, sem):
        wid = lax.axis_index("s") * info.num_cores + lax.axis_index("c")
        base = wid * bpw
        pltpu.sync_copy(idx_hbm.at[pl.ds(base, bpw)], idx_v)
        pltpu.sync_copy(emb_hbm.at[pl.ds(base, bpw)], e_v)
        pltpu.async_copy(table_hbm.at[idx_v], rows_v, sem).wait()

        lane = lax.broadcasted_iota(jnp.int32, (_L,), 0)

        def group(g, acc):
            rows16 = g * _L + lane
            accd = jnp.zeros((_L,), jnp.float32)
            accn = jnp.zeros((_L,), jnp.float32)
            for j in range(_D):
                col = jnp.full((_L,), j, jnp.int32)
                ev = plsc.load_gather(e_v, [rows16, col])
                pv = plsc.load_gather(rows_v, [rows16, col])
                accd = accd + ev * pv
                accn = accn + ev * ev
            return acc + accd * _nr_rsqrt(accn)

        acc = lax.fori_loop(0, ngroups, group, jnp.zeros((_L,), jnp.float32))
        acc_v[...] = acc
        pltpu.sync_copy(acc_v, out_hbm.at[wid])

    return pos_kernel(ptab_padded, emb_hbm_arg, pos_idx)


def _loss_body(emb_ref, protos_ref, out_ref, acc_ref):
    e = emb_ref[...]
    n = jnp.sqrt(jnp.sum(e * e, axis=1, keepdims=True))
    es = e * ((1.4426950408889634 / _TAU) / jnp.maximum(n, 1e-12))
    logits2 = lax.dot_general(
        es.astype(jnp.bfloat16), protos_ref[...], (((1,), (1,)), ((), ())),
        preferred_element_type=jnp.float32)
    denom = jnp.sum(jnp.exp2(logits2), axis=1)
    part = jnp.sum(jnp.log(denom + 1e-12))

    @pl.when(pl.program_id(0) == 0)
    def _():
        acc_ref[0] = 0.0

    acc_ref[0] += part

    @pl.when(pl.program_id(0) == pl.num_programs(0) - 1)
    def _():
        out_ref[0, 0] = acc_ref[0]


def _combine_body(dense_ref, sc_ref, out_ref):
    pos_sum = jnp.sum(sc_ref[...]) * (1.0 / _TAU)
    out_ref[0, 0] = (dense_ref[0, 0] - pos_sum) * (1.0 / _B)


def kernel(embeddings, positive_proto_ids, prototypes):
    sc_part = _sc_pos_term(prototypes, embeddings,
                           positive_proto_ids.astype(jnp.int32))
    grid = _B // _BLOCK_B
    dense = pl.pallas_call(
        _loss_body,
        grid=(grid,),
        in_specs=[
            pl.BlockSpec((_BLOCK_B, _D), lambda i: (i, 0)),
            pl.BlockSpec((_K, _D), lambda i: (0, 0)),
        ],
        out_specs=pl.BlockSpec(memory_space=pltpu.SMEM),
        out_shape=jax.ShapeDtypeStruct((1, 1), jnp.float32),
        scratch_shapes=[pltpu.SMEM((1,), jnp.float32)],
    )(embeddings, prototypes.astype(jnp.bfloat16))
    loss = pl.pallas_call(
        _combine_body,
        in_specs=[
            pl.BlockSpec(memory_space=pltpu.SMEM),
            pl.BlockSpec(memory_space=pltpu.VMEM),
        ],
        out_specs=pl.BlockSpec(memory_space=pltpu.SMEM),
        out_shape=jax.ShapeDtypeStruct((1, 1), jnp.float32),
    )(dense, sc_part)
    return loss[0, 0]


# speedup vs baseline: 1.0659x; 1.0659x over previous
import functools

import jax
import jax.numpy as jnp
from jax import lax
from jax.experimental import pallas as pl
from jax.experimental.pallas import tpu as pltpu
from jax.experimental.pallas import tpu_sc as plsc

_K = 8192
_D = 32
_B = 16384
_TAU = 0.07
_BLOCK_B = 1024
_DP = 128
_L = 16


def _nr_rsqrt(x):
    x = jnp.maximum(x, 1e-35)
    i = plsc.bitcast(x, jnp.int32)
    y = plsc.bitcast(jnp.int32(0x5F3759DF) - (i >> 1), jnp.float32)
    for _ in range(3):
        y = y * (1.5 - 0.5 * x * y * y)
    return y


def _sc_pos_term(ptab_padded, emb_hbm_arg, pos_idx):
    info = plsc.get_sparse_core_info()
    nw = info.num_cores * info.num_subcores
    bpw = _B // nw
    ngroups = bpw // _L
    mesh = plsc.VectorSubcoreMesh(core_axis_name="c", subcore_axis_name="s")

    @functools.partial(
        pl.kernel,
        mesh=mesh,
        out_type=jax.ShapeDtypeStruct((nw, _L), jnp.float32),
        compiler_params=pltpu.CompilerParams(needs_layout_passes=False),
        scratch_types=[
            pltpu.VMEM((4, bpw // 4), jnp.int32),
            pltpu.VMEM((bpw // 4, _DP), jnp.float32),
            pltpu.VMEM((bpw, _D), jnp.float32),
            pltpu.VMEM((_L,), jnp.float32),
            pltpu.SemaphoreType.DMA,
        ],
    )
    def pos_kernel(table_hbm, emb_hbm, idx_hbm, out_hbm,
                   idx_v, rows_v, e_v, acc_v, sem):
        wid = lax.axis_index("s") * info.num_cores + lax.axis_index("c")
        base = wid * bpw
        qtr = bpw // 4
        pltpu.sync_copy(emb_hbm.at[pl.ds(base, bpw)], e_v)
        for c in range(4):
            pltpu.sync_copy(
                idx_hbm.at[pl.ds(base + c * qtr, qtr)], idx_v.at[c])

        lane = lax.broadcasted_iota(jnp.int32, (_L,), 0)
        acc = jnp.zeros((_L,), jnp.float32)
        for c in range(4):
            pltpu.async_copy(table_hbm.at[idx_v.at[c]], rows_v, sem).wait()

            def group(g, acc, _c=c):
                rows16 = g * _L + lane
                accd = jnp.zeros((_L,), jnp.float32)
                accn = jnp.zeros((_L,), jnp.float32)
                for j in range(_D):
                    col = jnp.full((_L,), j, jnp.int32)
                    ev = plsc.load_gather(e_v, [_c * qtr + rows16, col])
                    pv = plsc.load_gather(rows_v, [rows16, col])
                    accd = accd + ev * pv
                    accn = accn + ev * ev
                return acc + accd * _nr_rsqrt(accn)

            acc = lax.fori_loop(0, qtr // _L, group, acc)
        acc_v[...] = acc
        pltpu.sync_copy(acc_v, out_hbm.at[wid])

    return pos_kernel(ptab_padded, emb_hbm_arg, pos_idx)


def _loss_body(emb_ref, protos_ref, out_ref, acc_ref):
    e = emb_ref[...]
    n = jnp.sqrt(jnp.sum(e * e, axis=1, keepdims=True))
    es = e * ((1.0 / _TAU) / jnp.maximum(n, 1e-12))
    logits = lax.dot_general(
        es.astype(jnp.bfloat16), protos_ref[...], (((1,), (1,)), ((), ())),
        preferred_element_type=jnp.float32)
    denom = jnp.sum(jnp.exp(logits), axis=1)
    part = jnp.sum(jnp.log(denom + 1e-12))

    @pl.when(pl.program_id(0) == 0)
    def _():
        acc_ref[0] = 0.0

    acc_ref[0] += part

    @pl.when(pl.program_id(0) == pl.num_programs(0) - 1)
    def _():
        out_ref[0, 0] = acc_ref[0]


def _combine_body(dense_ref, sc_ref, out_ref):
    pos_sum = jnp.sum(sc_ref[...]) * (1.0 / _TAU)
    out_ref[0, 0] = (dense_ref[0, 0] - pos_sum) * (1.0 / _B)


def kernel(embeddings, positive_proto_ids, prototypes):
    ptab = jnp.pad(prototypes, ((0, 0), (0, _DP - _D)))
    sc_part = _sc_pos_term(ptab, embeddings,
                           positive_proto_ids.astype(jnp.int32))
    grid = _B // _BLOCK_B
    dense = pl.pallas_call(
        _loss_body,
        grid=(grid,),
        in_specs=[
            pl.BlockSpec((_BLOCK_B, _D), lambda i: (i, 0)),
            pl.BlockSpec((_K, _D), lambda i: (0, 0)),
        ],
        out_specs=pl.BlockSpec(memory_space=pltpu.SMEM),
        out_shape=jax.ShapeDtypeStruct((1, 1), jnp.float32),
        scratch_shapes=[pltpu.SMEM((1,), jnp.float32)],
    )(embeddings, prototypes.astype(jnp.bfloat16))
    loss = pl.pallas_call(
        _combine_body,
        in_specs=[
            pl.BlockSpec(memory_space=pltpu.SMEM),
            pl.BlockSpec(memory_space=pltpu.VMEM),
        ],
        out_specs=pl.BlockSpec(memory_space=pltpu.SMEM),
        out_shape=jax.ShapeDtypeStruct((1, 1), jnp.float32),
    )(dense, sc_part)
    return loss[0, 0]


# speedup vs baseline: 1.1106x; 1.0419x over previous
import functools

import jax
import jax.numpy as jnp
from jax import lax
from jax.experimental import pallas as pl
from jax.experimental.pallas import tpu as pltpu
from jax.experimental.pallas import tpu_sc as plsc

_K = 8192
_D = 32
_B = 16384
_TAU = 0.07
_BLOCK_B = 2048
_DP = 128
_L = 16


def _nr_rsqrt(x):
    x = jnp.maximum(x, 1e-35)
    i = plsc.bitcast(x, jnp.int32)
    y = plsc.bitcast(jnp.int32(0x5F3759DF) - (i >> 1), jnp.float32)
    for _ in range(3):
        y = y * (1.5 - 0.5 * x * y * y)
    return y


def _sc_pos_term(ptab_padded, emb_hbm_arg, pos_idx):
    info = plsc.get_sparse_core_info()
    nw = info.num_cores * info.num_subcores
    bpw = _B // nw
    ngroups = bpw // _L
    mesh = plsc.VectorSubcoreMesh(core_axis_name="c", subcore_axis_name="s")

    @functools.partial(
        pl.kernel,
        mesh=mesh,
        out_type=jax.ShapeDtypeStruct((nw, _L), jnp.float32),
        compiler_params=pltpu.CompilerParams(needs_layout_passes=False),
        scratch_types=[
            pltpu.VMEM((4, bpw // 4), jnp.int32),
            pltpu.VMEM((bpw // 4, _DP), jnp.float32),
            pltpu.VMEM((bpw, _D), jnp.float32),
            pltpu.VMEM((_L,), jnp.float32),
            pltpu.SemaphoreType.DMA,
        ],
    )
    def pos_kernel(table_hbm, emb_hbm, idx_hbm, out_hbm,
                   idx_v, rows_v, e_v, acc_v, sem):
        wid = lax.axis_index("s") * info.num_cores + lax.axis_index("c")
        base = wid * bpw
        qtr = bpw // 4
        pltpu.sync_copy(emb_hbm.at[pl.ds(base, bpw)], e_v)
        for c in range(4):
            pltpu.sync_copy(
                idx_hbm.at[pl.ds(base + c * qtr, qtr)], idx_v.at[c])

        lane = lax.broadcasted_iota(jnp.int32, (_L,), 0)
        acc = jnp.zeros((_L,), jnp.float32)
        for c in range(4):
            pltpu.async_copy(table_hbm.at[idx_v.at[c]], rows_v, sem).wait()

            def group(g, acc, _c=c):
                rows16 = g * _L + lane
                accd = jnp.zeros((_L,), jnp.float32)
                accn = jnp.zeros((_L,), jnp.float32)
                for j in range(_D):
                    col = jnp.full((_L,), j, jnp.int32)
                    ev = plsc.load_gather(e_v, [_c * qtr + rows16, col])
                    pv = plsc.load_gather(rows_v, [rows16, col])
                    accd = accd + ev * pv
                    accn = accn + ev * ev
                return acc + accd * _nr_rsqrt(accn)

            acc = lax.fori_loop(0, qtr // _L, group, acc)
        acc_v[...] = acc
        pltpu.sync_copy(acc_v, out_hbm.at[wid])

    return pos_kernel(ptab_padded, emb_hbm_arg, pos_idx)


def _loss_body(emb_ref, protos_ref, out_ref, acc_ref):
    e = emb_ref[...]
    n = jnp.sqrt(jnp.sum(e * e, axis=1, keepdims=True))
    es = e * ((1.0 / _TAU) / jnp.maximum(n, 1e-12))
    logits = lax.dot_general(
        es.astype(jnp.bfloat16), protos_ref[...], (((1,), (1,)), ((), ())),
        preferred_element_type=jnp.float32)
    denom = jnp.sum(jnp.exp(logits), axis=1)
    part = jnp.sum(jnp.log(denom + 1e-12))

    @pl.when(pl.program_id(0) == 0)
    def _():
        acc_ref[0] = 0.0

    acc_ref[0] += part

    @pl.when(pl.program_id(0) == pl.num_programs(0) - 1)
    def _():
        out_ref[0, 0] = acc_ref[0]


def _combine_body(dense_ref, sc_ref, out_ref):
    pos_sum = jnp.sum(sc_ref[...]) * (1.0 / _TAU)
    out_ref[0, 0] = (dense_ref[0, 0] - pos_sum) * (1.0 / _B)


def kernel(embeddings, positive_proto_ids, prototypes):
    ptab = jnp.pad(prototypes, ((0, 0), (0, _DP - _D)))
    sc_part = _sc_pos_term(ptab, embeddings,
                           positive_proto_ids.astype(jnp.int32))
    grid = _B // _BLOCK_B
    dense = pl.pallas_call(
        _loss_body,
        grid=(grid,),
        in_specs=[
            pl.BlockSpec((_BLOCK_B, _D), lambda i: (i, 0)),
            pl.BlockSpec((_K, _D), lambda i: (0, 0)),
        ],
        out_specs=pl.BlockSpec(memory_space=pltpu.SMEM),
        out_shape=jax.ShapeDtypeStruct((1, 1), jnp.float32),
        scratch_shapes=[pltpu.SMEM((1,), jnp.float32)],
        compiler_params=pltpu.CompilerParams(
            vmem_limit_bytes=128 * 1024 * 1024),
    )(embeddings, prototypes.astype(jnp.bfloat16))
    loss = pl.pallas_call(
        _combine_body,
        in_specs=[
            pl.BlockSpec(memory_space=pltpu.SMEM),
            pl.BlockSpec(memory_space=pltpu.VMEM),
        ],
        out_specs=pl.BlockSpec(memory_space=pltpu.SMEM),
        out_shape=jax.ShapeDtypeStruct((1, 1), jnp.float32),
    )(dense, sc_part)
    return loss[0, 0]
